# Initial kernel scaffold; baseline (speedup 1.0000x reference)
import jax
import jax.numpy as jnp
from jax.experimental import pallas as pl


def kernel(rois, features, stride):
    raise NotImplementedError("write your pallas kernel here")


# speedup vs baseline: 26.5556x; 26.5556x over previous
import functools

import jax
import jax.numpy as jnp
from jax import lax
from jax.experimental import pallas as pl
from jax.experimental.pallas import tpu as pltpu
from jax.experimental.pallas import tpu_sc as plsc

GS = 7
WPAD = 72


def _integral_body(f_ref, out_ref):
    h = f_ref.shape[2]
    ri = lax.broadcasted_iota(jnp.int32, (h, h), 0)
    ci = lax.broadcasted_iota(jnp.int32, (h, h), 1)
    ltri = (ri >= ci).astype(jnp.float32)
    utri = (ri <= ci).astype(jnp.float32)

    out_ref[...] = jnp.zeros(out_ref.shape, jnp.float32)

    def per_channel(i, carry):
        x = f_ref[0, i, :, :]
        cw = jnp.dot(x, utri, preferred_element_type=jnp.float32)
        chw = jnp.dot(ltri, cw, preferred_element_type=jnp.float32)
        out_ref[0, i, 0, pl.ds(1, h), pl.ds(1, h)] = chw
        return carry

    lax.fori_loop(0, f_ref.shape[1], per_channel, 0)


def _integral_image(features, od):
    b, c, h, w = features.shape
    return pl.pallas_call(
        _integral_body,
        grid=(b, od),
        in_specs=[pl.BlockSpec((1, GS * GS, h, w), lambda bi, oi: (bi, oi, 0, 0))],
        out_specs=pl.BlockSpec((1, GS * GS, 1, h + 1, WPAD),
                               lambda bi, oi: (oi, 0, bi, 0, 0)),
        out_shape=jax.ShapeDtypeStruct((od, GS * GS, b, h + 1, WPAD), jnp.float32),
    )(features)


def _psroi_sc(rois_flat, iview, n, npad, od, hdim):
    og = od // GS
    ntask = GS * GS * og
    nw = 32
    kmax = -(-ntask // nw)
    nchunk = npad // 16
    rpad = -(-(n * 5) // 16) * 16 + 16
    mesh = plsc.VectorSubcoreMesh(core_axis_name="c", subcore_axis_name="s")
    hlim = float(hdim)

    @functools.partial(
        pl.kernel,
        mesh=mesh,
        compiler_params=pltpu.CompilerParams(
            needs_layout_passes=False, use_tc_tiling_on_sc=False),
        out_type=jax.ShapeDtypeStruct((ntask, npad, GS), jnp.float32),
        scratch_types=[
            pltpu.VMEM((rpad,), jnp.float32),
            pltpu.VMEM((GS, 2, hdim + 1, WPAD), jnp.float32),
            pltpu.VMEM((npad, GS), jnp.float32),
        ],
    )
    def body(rois_hbm, iview_hbm, out_hbm, rois_v, table_v, out_v):
        wid = lax.axis_index("s") * 2 + lax.axis_index("c")

        def zero(i, carry):
            rois_v[pl.ds(i * 16, 16)] = jnp.zeros((16,), jnp.float32)
            return carry

        lax.fori_loop(0, rpad // 16, zero, 0)
        pltpu.sync_copy(rois_hbm, rois_v.at[pl.ds(0, n * 5)])

        def run_task(t):
            p = t // og
            g = t - p * og
            gh = p // GS
            gw = p - gh * GS
            pltpu.sync_copy(iview_hbm.at[pl.ds(g * GS, GS), p], table_v)

            ghf = gh.astype(jnp.float32)
            gwf = gw.astype(jnp.float32)

            def chunk(ci, carry):
                base = ci * 16
                nvec = base + lax.broadcasted_iota(jnp.int32, (16,), 0)
                f0 = nvec * 5
                bf = plsc.load_gather(rois_v, [f0])
                rsw = plsc.load_gather(rois_v, [f0 + 1])
                rsh = plsc.load_gather(rois_v, [f0 + 2])
                rew = plsc.load_gather(rois_v, [f0 + 3])
                reh = plsc.load_gather(rois_v, [f0 + 4])
                bi = jnp.clip(bf.astype(jnp.int32), 0, 1)
                binw = jnp.maximum(rew - rsw, 0.1) / GS
                binh = jnp.maximum(reh - rsh, 0.1) / GS
                hsf = jnp.clip(ghf * binh + rsh, 0.0, hlim)
                wsf = jnp.clip(gwf * binw + rsw, 0.0, hlim)
                hef = jnp.clip((ghf + 1.0) * binh + rsh, 0.0, hlim)
                wef = jnp.clip((gwf + 1.0) * binw + rsw, 0.0, hlim)
                hs = hsf.astype(jnp.int32)
                ws = wsf.astype(jnp.int32)
                he = hef.astype(jnp.int32)
                he = he + (he.astype(jnp.float32) < hef).astype(jnp.int32)
                we = wef.astype(jnp.int32)
                we = we + (we.astype(jnp.float32) < wef).astype(jnp.int32)
                area = (he - hs) * (we - ws)
                inv = 1.0 / jnp.maximum(area, 1).astype(jnp.float32)
                pos = area > 0
                for ol in range(GS):
                    ov = jnp.full((16,), ol, jnp.int32)
                    s = (plsc.load_gather(table_v, [ov, bi, he, we])
                         - plsc.load_gather(table_v, [ov, bi, hs, we])
                         - plsc.load_gather(table_v, [ov, bi, he, ws])
                         + plsc.load_gather(table_v, [ov, bi, hs, ws]))
                    val = jnp.where(pos, s * inv, jnp.float32(0.0))
                    plsc.store_scatter(out_v, [nvec, ov], val)
                return carry

            lax.fori_loop(0, nchunk, chunk, 0)
            pltpu.sync_copy(out_v, out_hbm.at[t])

        for k in range(kmax):
            t = wid + k * nw
            if (k + 1) * nw > ntask:
                @pl.when(t < ntask)
                def _():
                    run_task(t)
            else:
                run_task(t)

    return body(rois_flat, iview)


def kernel(rois, features, stride):
    b, c, h, w = features.shape
    n = rois.shape[0]
    od = c // (GS * GS)
    npad = -(-n // 16) * 16
    og = od // GS
    scale = 1.0 / jnp.asarray(stride, jnp.float32)
    r = rois[:, :5].astype(jnp.float32)
    rois_flat = jnp.stack(
        [r[:, 0],
         jnp.round(r[:, 1]) * scale,
         jnp.round(r[:, 2]) * scale,
         jnp.round(r[:, 3] + 1.0) * scale,
         jnp.round(r[:, 4] + 1.0) * scale],
        axis=1).reshape(-1)
    iview = _integral_image(features.astype(jnp.float32), od)
    blocks = _psroi_sc(rois_flat, iview, n, npad, od, h)
    out = blocks.reshape(GS * GS, og, npad, GS)
    out = jnp.transpose(out, (2, 1, 3, 0)).reshape(npad, od, GS * GS)
    return out[:n].reshape(n, od, GS, GS)
